# baseline (speedup 1.0000x reference)
import functools

import jax
import jax.numpy as jnp
from jax import lax
from jax.experimental import pallas as pl
from jax.experimental.pallas import tpu as pltpu
from jax.experimental.pallas import tpu_sc as plsc

B = 16384
F = 32
L = 16
NC = 2
NS = 16
NW = NC * NS
BPW = B // NW

_mesh = plsc.VectorSubcoreMesh(core_axis_name="c", subcore_axis_name="s")

_GATHER_DNUMS = lax.GatherDimensionNumbers(
    offset_dims=(), collapsed_slice_dims=(0,), start_index_map=(0,))


def _shuffle(v, idx):
    return lax.gather(v, idx[:, None], _GATHER_DNUMS, (1,),
                      mode=lax.GatherScatterMode.PROMISE_IN_BOUNDS)


@functools.partial(
    pl.kernel,
    mesh=_mesh,
    compiler_params=pltpu.CompilerParams(use_tc_tiling_on_sc=False),
    out_type=jax.ShapeDtypeStruct((B,), jnp.float32),
    scratch_types=[
        pltpu.VMEM((BPW,), jnp.int32),
        pltpu.VMEM((BPW,), jnp.int32),
        pltpu.VMEM((BPW, F), jnp.float32),
        pltpu.VMEM((BPW, F), jnp.float32),
        pltpu.VMEM((BPW,), jnp.float32),
        pltpu.SemaphoreType.DMA,
        pltpu.SemaphoreType.DMA,
    ],
)
def _gmf_kernel(uids_hbm, iids_hbm, utab_hbm, itab_hbm, out_hbm,
                uid_v, iid_v, ur_v, ir_v, o_v, usem, isem):
    wid = lax.axis_index("s") * NC + lax.axis_index("c")
    base = wid * BPW

    pltpu.sync_copy(uids_hbm.at[pl.ds(base, BPW)], uid_v)
    pltpu.sync_copy(iids_hbm.at[pl.ds(base, BPW)], iid_v)
    cu = pltpu.async_copy(utab_hbm.at[uid_v], ur_v, usem)
    ci = pltpu.async_copy(itab_hbm.at[iid_v], ir_v, isem)
    cu.wait()
    ci.wait()

    lanes = lax.iota(jnp.int32, L)
    rolls = [(lanes + d) & (L - 1) for d in (8, 4, 2, 1)]

    def group_body(g, carry):
        r0 = g * L
        acc = jnp.zeros((L,), jnp.float32)
        for k in range(L):
            r = r0 + k
            u0 = ur_v[r, pl.ds(0, L)]
            u1 = ur_v[r, pl.ds(L, L)]
            i0 = ir_v[r, pl.ds(0, L)]
            i1 = ir_v[r, pl.ds(L, L)]
            s = u0 * i0 + u1 * i1
            for rr in rolls:
                s = s + _shuffle(s, rr)
            acc = jnp.where(lanes == k, s, acc)
        o_v[pl.ds(r0, L)] = 1.0 / (1.0 + jnp.exp(-acc))
        return carry

    lax.fori_loop(0, BPW // L, group_body, 0)

    pltpu.sync_copy(o_v, out_hbm.at[pl.ds(base, BPW)])


def kernel(user_ids, item_ids, user_table, item_table):
    return _gmf_kernel(user_ids.astype(jnp.int32), item_ids.astype(jnp.int32),
                       user_table, item_table)
